# Initial kernel scaffold; baseline (speedup 1.0000x reference)
import jax
import jax.numpy as jnp
from jax.experimental import pallas as pl


def kernel(labels, edge_src_task, edge_dst_wkr, edge_type, weight_worker, weight_task):
    raise NotImplementedError("write your pallas kernel here")


# speedup vs baseline: 4.4139x; 4.4139x over previous
import functools
import jax
import jax.numpy as jnp
from jax import lax
from jax.experimental import pallas as pl
from jax.experimental.pallas import tpu as pltpu
from jax.experimental.pallas import tpu_sc as plsc

F32 = jnp.float32
I32 = jnp.int32

N_WKR = 2000
N_TSK = 8000
R = 10
D = 256
E = 160000

NC = 2
NS = 16
NT = NC * NS
CH = 128
EPT = 5120
E_PAD = NT * EPT
NCH = EPT // CH

WKR_PAD = 2048
TSK_PAD = 8064
WSTRIPE = WKR_PAD // NS
TSTRIPE = TSK_PAD // NS
ZROWS = TSTRIPE // 3


def _mm_a_body(lab_ref, w_ref, out_ref):
    out_ref[0] = jnp.dot(lab_ref[...], w_ref[0],
                         preferred_element_type=F32)


def _transform_tasks(labels, weight_worker):
    tb = 2000
    return pl.pallas_call(
        _mm_a_body,
        grid=(R, N_TSK // tb),
        in_specs=[
            pl.BlockSpec((tb, D), lambda r, t: (t, 0)),
            pl.BlockSpec((1, D, D), lambda r, t: (r, 0, 0)),
        ],
        out_specs=pl.BlockSpec((1, tb, D), lambda r, t: (r, t, 0)),
        out_shape=jax.ShapeDtypeStruct((R, N_TSK, D), F32),
    )(labels, weight_worker)


def _mm_c_body(accA_ref, degW_ref, w_ref, abil_ref, hw_ref):
    sumA = accA_ref[0] + accA_ref[1]
    deg = jnp.maximum(jnp.sum(degW_ref[...], axis=0), 1.0)[:, None]
    ability = sumA / deg
    abil_ref[...] = ability
    hw_ref[0] = jnp.dot(ability, w_ref[0], preferred_element_type=F32)


def _transform_workers(accA_p, degW_p, weight_task):
    return pl.pallas_call(
        _mm_c_body,
        grid=(R,),
        in_specs=[
            pl.BlockSpec((NC, WKR_PAD, D), lambda r: (0, 0, 0)),
            pl.BlockSpec((NT, WKR_PAD), lambda r: (0, 0)),
            pl.BlockSpec((1, D, D), lambda r: (r, 0, 0)),
        ],
        out_specs=[
            pl.BlockSpec((WKR_PAD, D), lambda r: (0, 0)),
            pl.BlockSpec((1, WKR_PAD, D), lambda r: (r, 0, 0)),
        ],
        out_shape=[
            jax.ShapeDtypeStruct((WKR_PAD, D), F32),
            jax.ShapeDtypeStruct((R, WKR_PAD, D), F32),
        ],
    )(accA_p, degW_p, weight_task)


def _norm_body(acc_ref, degT_ref, out_ref):
    s = acc_ref[0] + acc_ref[1]
    deg = jnp.maximum(jnp.sum(degT_ref[...], axis=0), 1.0)[:, None]
    out_ref[...] = s / deg


def _normalize_tasks(acc2_p, degT_p):
    nb = 7
    rb = TSK_PAD // nb
    return pl.pallas_call(
        _norm_body,
        grid=(nb,),
        in_specs=[
            pl.BlockSpec((NC, rb, D), lambda i: (0, i, 0)),
            pl.BlockSpec((NT, rb), lambda i: (0, i)),
        ],
        out_specs=pl.BlockSpec((rb, D), lambda i: (i, 0)),
        out_shape=jax.ShapeDtypeStruct((TSK_PAD, D), F32),
    )(acc2_p, degT_p)


def _sc_mesh():
    return plsc.VectorSubcoreMesh(core_axis_name="c", subcore_axis_name="s",
                                  num_cores=NC, num_subcores=NS)


def _zero_1d(ref):
    z = jnp.zeros((16,), F32)
    def body(i, _):
        ref[pl.ds(i * 16, 16)] = z
        return 0
    lax.fori_loop(0, ref.shape[0] // 16, body, 0)


def _phase1_sc(esrc, edst, etyp, table, zbig):
    @functools.partial(
        pl.kernel,
        out_type=[
            jax.ShapeDtypeStruct((NC, WKR_PAD, D), F32),
            jax.ShapeDtypeStruct((NT, WKR_PAD), F32),
            jax.ShapeDtypeStruct((NT, TSK_PAD), F32),
        ],
        mesh=_sc_mesh(),
        compiler_params=pltpu.CompilerParams(needs_layout_passes=False),
        scratch_types=[
            pltpu.VMEM((CH,), I32),
            pltpu.VMEM((CH,), I32),
            pltpu.VMEM((CH,), I32),
            pltpu.VMEM((CH,), I32),
            pltpu.VMEM((CH, D), F32),
            pltpu.VMEM((WKR_PAD,), F32),
            pltpu.VMEM((TSK_PAD,), F32),
            pltpu.VMEM((ZROWS, D), F32),
            pltpu.SemaphoreType.DMA,
        ],
    )
    def k(esrc_h, edst_h, etyp_h, table_h, zbig_h,
          accA_o, degW_o, degT_o,
          srcv, dstv, typv, gidxv, rows, degW, degT, zbuf, sem):
        cid = lax.axis_index("c")
        sid = lax.axis_index("s")
        wid = sid * NC + cid

        pltpu.sync_copy(zbig_h.at[pl.ds(0, ZROWS)], zbuf)
        pltpu.sync_copy(zbuf.at[pl.ds(0, WSTRIPE)],
                        accA_o.at[cid, pl.ds(sid * WSTRIPE, WSTRIPE)])
        _zero_1d(degW)
        _zero_1d(degT)
        plsc.subcore_barrier()

        ones16 = jnp.ones((16,), F32)

        def chunk(c, _):
            b = wid * EPT + c * CH
            pltpu.sync_copy(esrc_h.at[pl.ds(b, CH)], srcv)
            pltpu.sync_copy(edst_h.at[pl.ds(b, CH)], dstv)
            pltpu.sync_copy(etyp_h.at[pl.ds(b, CH)], typv)

            def vec(i, _):
                s = srcv[pl.ds(i * 16, 16)]
                d = dstv[pl.ds(i * 16, 16)]
                t = typv[pl.ds(i * 16, 16)]
                gidxv[pl.ds(i * 16, 16)] = (
                    t * N_TSK + jnp.minimum(s, N_TSK - 1))
                plsc.addupdate_scatter(degW, [d], ones16)
                plsc.addupdate_scatter(degT, [s], ones16)
                return 0
            lax.fori_loop(0, CH // 16, vec, 0)

            pltpu.async_copy(table_h.at[gidxv], rows, sem).wait()
            pltpu.sync_copy(rows, accA_o.at[cid].at[dstv], add=True)
            return 0
        lax.fori_loop(0, NCH, chunk, 0)

        pltpu.sync_copy(degW, degW_o.at[wid])
        pltpu.sync_copy(degT, degT_o.at[wid])

    return k(esrc, edst, etyp, table, zbig)


def _phase2_sc(esrc, edst, etyp, table, zbig):
    @functools.partial(
        pl.kernel,
        out_type=jax.ShapeDtypeStruct((NC, TSK_PAD, D), F32),
        mesh=_sc_mesh(),
        compiler_params=pltpu.CompilerParams(needs_layout_passes=False),
        scratch_types=[
            pltpu.VMEM((CH,), I32),
            pltpu.VMEM((CH,), I32),
            pltpu.VMEM((CH,), I32),
            pltpu.VMEM((CH,), I32),
            pltpu.VMEM((CH, D), F32),
            pltpu.VMEM((ZROWS, D), F32),
            pltpu.SemaphoreType.DMA,
        ],
    )
    def k(esrc_h, edst_h, etyp_h, table_h, zbig_h,
          acc_o, srcv, dstv, typv, gidxv, rows, zbuf, sem):
        cid = lax.axis_index("c")
        sid = lax.axis_index("s")
        wid = sid * NC + cid

        pltpu.sync_copy(zbig_h.at[pl.ds(0, ZROWS)], zbuf)
        for j in range(3):
            pltpu.sync_copy(
                zbuf,
                acc_o.at[cid, pl.ds(sid * TSTRIPE + j * ZROWS, ZROWS)])
        plsc.subcore_barrier()

        def chunk(c, _):
            b = wid * EPT + c * CH
            pltpu.sync_copy(esrc_h.at[pl.ds(b, CH)], srcv)
            pltpu.sync_copy(edst_h.at[pl.ds(b, CH)], dstv)
            pltpu.sync_copy(etyp_h.at[pl.ds(b, CH)], typv)

            def vec(i, _):
                d = dstv[pl.ds(i * 16, 16)]
                t = typv[pl.ds(i * 16, 16)]
                gidxv[pl.ds(i * 16, 16)] = (
                    t * WKR_PAD + jnp.minimum(d, WKR_PAD - 1))
                return 0
            lax.fori_loop(0, CH // 16, vec, 0)

            pltpu.async_copy(table_h.at[gidxv], rows, sem).wait()
            pltpu.sync_copy(rows, acc_o.at[cid].at[srcv], add=True)
            return 0
        lax.fori_loop(0, NCH, chunk, 0)

    return k(esrc, edst, etyp, table, zbig)


@jax.jit
def kernel(labels, edge_src_task, edge_dst_wkr, edge_type,
           weight_worker, weight_task):
    npad = E_PAD - E
    esrc = jnp.concatenate(
        [edge_src_task.astype(I32), jnp.full((npad,), N_TSK, I32)])
    edst = jnp.concatenate(
        [edge_dst_wkr.astype(I32), jnp.full((npad,), N_WKR, I32)])
    etyp = jnp.concatenate(
        [edge_type.astype(I32), jnp.zeros((npad,), I32)])
    zbig = jnp.zeros((TSK_PAD, D), F32)

    ht = _transform_tasks(labels, weight_worker).reshape(R * N_TSK, D)
    accA_p, degW_p, degT_p = _phase1_sc(esrc, edst, etyp, ht, zbig)
    ability_full, hw = _transform_workers(accA_p, degW_p, weight_task)
    acc2_p = _phase2_sc(esrc, edst, etyp, hw.reshape(R * WKR_PAD, D), zbig)
    new_full = _normalize_tasks(acc2_p, degT_p)
    return ability_full[:N_WKR], new_full[:N_TSK]
